# baseline (speedup 1.0000x reference)
import jax
import jax.numpy as jnp
from jax.experimental import pallas as pl
from jax.experimental.pallas import tpu as pltpu


_B = 16384
_D = 64
_NCHUNK = 32
_CH = _B // _NCHUNK


def _stream_dot_kernel(x_hbm, o_ref, bufs, sems):

    def copy(c):
        return pltpu.make_async_copy(
            x_hbm.at[:, :, pl.ds(c * _CH, _CH)],
            bufs.at[c],
            sems.at[c],
        )

    for c in range(_NCHUNK):
        copy(c).start()
    for c in range(_NCHUNK):
        copy(c).wait()
        prod = bufs[c, 0] * bufs[c, 1]
        o_ref[pl.ds(c * _CH, _CH)] = jnp.sum(prod, axis=0)


def kernel(inputs):
    xt = jnp.transpose(inputs, (0, 2, 1))
    return pl.pallas_call(
        _stream_dot_kernel,
        in_specs=[pl.BlockSpec(memory_space=pltpu.MemorySpace.HBM)],
        out_specs=pl.BlockSpec(memory_space=pltpu.VMEM),
        out_shape=jax.ShapeDtypeStruct((_B,), jnp.float32),
        scratch_shapes=[
            pltpu.VMEM((_NCHUNK, 2, _D, _CH), jnp.float32),
            pltpu.SemaphoreType.DMA((_NCHUNK,)),
        ],
    )(xt)


# speedup vs baseline: 1.0268x; 1.0268x over previous
import jax
import jax.numpy as jnp
from jax.experimental import pallas as pl
from jax.experimental.pallas import tpu as pltpu


_B = 16384
_D = 64
_NCHUNK = 16
_CH = _B // _NCHUNK


def _stream_dot_kernel(x_hbm, o_ref, bufs, sems):

    def copy(c):
        return pltpu.make_async_copy(
            x_hbm.at[:, :, pl.ds(c * _CH, _CH)],
            bufs.at[c],
            sems.at[c],
        )

    for c in range(_NCHUNK):
        copy(c).start()
    for c in range(_NCHUNK):
        copy(c).wait()
        prod = bufs[c, 0] * bufs[c, 1]
        o_ref[pl.ds(c * _CH, _CH)] = jnp.sum(prod, axis=0)


def kernel(inputs):
    xt = jnp.transpose(inputs, (0, 2, 1))
    return pl.pallas_call(
        _stream_dot_kernel,
        in_specs=[pl.BlockSpec(memory_space=pltpu.MemorySpace.HBM)],
        out_specs=pl.BlockSpec(memory_space=pltpu.VMEM),
        out_shape=jax.ShapeDtypeStruct((_B,), jnp.float32),
        scratch_shapes=[
            pltpu.VMEM((_NCHUNK, 2, _D, _CH), jnp.float32),
            pltpu.SemaphoreType.DMA((_NCHUNK,)),
        ],
    )(xt)
